# baseline (speedup 1.0000x reference)
import functools
import jax
import jax.numpy as jnp
from jax import lax
from jax.experimental import pallas as pl
from jax.experimental.pallas import tpu as pltpu
from jax.experimental.pallas import tpu_sc as plsc

_N = 16384
_K = 128
_NUM_SUBCORES = 16
_CHUNK = _N // _NUM_SUBCORES
_LANES = 16

_mesh = plsc.VectorSubcoreMesh(core_axis_name="c", subcore_axis_name="s", num_cores=1)


@functools.partial(
    pl.kernel,
    out_type=jax.ShapeDtypeStruct((_N,), jnp.int32),
    mesh=_mesh,
    scratch_types=[
        pltpu.VMEM((_CHUNK,), jnp.int32),
        pltpu.VMEM((_K,), jnp.int32),
        pltpu.VMEM((_K,), jnp.int32),
        pltpu.VMEM((_CHUNK,), jnp.int32),
        pltpu.SemaphoreType.DMA,
        pltpu.SemaphoreType.DMA,
    ],
    compiler_params=pltpu.CompilerParams(
        needs_layout_passes=False,
    ),
)
def _lookup(x_hbm, cond_hbm, out_hbm, x_v, cond_v, inv_v, out_v, sem_x, sem_c):
    wid = lax.axis_index("s")
    base = wid * _CHUNK
    cp_x = pltpu.async_copy(x_hbm.at[pl.ds(base, _CHUNK)], x_v, sem_x)
    cp_c = pltpu.async_copy(cond_hbm, cond_v, sem_c)
    cp_c.wait()
    lane = lax.iota(jnp.int32, _LANES)
    @plsc.parallel_loop(0, _K, step=_LANES)
    def build(off):
        vals = cond_v[pl.ds(off, _LANES)]
        plsc.store_scatter(inv_v, [vals], lane + off)
    cp_x.wait()
    @plsc.parallel_loop(0, _CHUNK, step=_LANES, unroll=2)
    def body(off):
        query = x_v[pl.ds(off, _LANES)]
        out_v[pl.ds(off, _LANES)] = plsc.load_gather(inv_v, [query])
    pltpu.sync_copy(out_v, out_hbm.at[pl.ds(base, _CHUNK)])


def kernel(x, condition_tensors):
    idx = _lookup(x, condition_tensors)
    return idx.reshape(-1, 1, 1).astype(jnp.int64)
